# baseline (speedup 1.0000x reference)
import functools

import jax
import jax.numpy as jnp
from jax import lax
from jax.experimental import pallas as pl
from jax.experimental.pallas import tpu as pltpu

_EPS = 1e-5
_NUM_STEPS = 8
_GROUP = 4
_TPAD = 16


def _ode_kernel(x_ref, w1_ref, w2_ref, valid_ref, cols_ref, out_ref,
                *, spatial_w, num_steps, channels):
    f32 = jnp.float32
    bf16 = jnp.bfloat16
    C = channels
    GC = out_ref.shape[1]
    G = GC // C
    HW = out_ref.shape[2]
    Wd = spatial_w

    w1 = w1_ref[...]
    w2 = w2_ref[...]
    valid = valid_ref[...]
    cols = cols_ref[...]
    g1w, g1b = cols[:, 0:1], cols[:, 1:2]
    g2w, g2b = cols[:, 2:3], cols[:, 3:4]
    g3w, g3b = cols[:, 4:5], cols[:, 5:6]

    wcol = lax.broadcasted_iota(jnp.int32, (1, HW), 1) % Wd
    mask_l = (wcol >= 1).astype(bf16)
    mask_r = (wcol < Wd - 1).astype(bf16)

    def shift(h, off):
        if off == 0:
            return h
        pad = jnp.zeros((h.shape[0], abs(off)), h.dtype)
        if off > 0:
            return jnp.concatenate([h[:, off:], pad], axis=1)
        return jnp.concatenate([pad, h[:, :off]], axis=1)

    def tap_block(hb, trows):
        rows = []
        for dh in (-1, 0, 1):
            base = shift(hb, dh * Wd)
            rows.append(shift(base, -1) * mask_l)
            rows.append(base)
            rows.append(shift(base, 1) * mask_r)
        rows.append(trows)
        return jnp.concatenate(rows, axis=0)

    def gn_relu_bf16(h, gw, gb):
        m = jnp.mean(h, axis=1, keepdims=True)
        ms = jnp.mean(h * h, axis=1, keepdims=True)
        scale = lax.rsqrt((ms - m * m) + _EPS) * gw
        return jnp.maximum(h * scale + (gb - m * scale), 0.0).astype(bf16)

    def gn_final(h, gw, gb):
        m = jnp.mean(h, axis=1, keepdims=True)
        ms = jnp.mean(h * h, axis=1, keepdims=True)
        scale = lax.rsqrt((ms - m * m) + _EPS) * gw
        return h * scale + (gb - m * scale)

    def conv(hb, w, t):
        trows = (t * valid).astype(bf16)
        outs = [jnp.dot(w, tap_block(hb[g * C:(g + 1) * C], trows),
                        preferred_element_type=f32) for g in range(G)]
        return outs[0] if G == 1 else jnp.concatenate(outs, axis=0)

    def odefunc(t, y):
        h = gn_relu_bf16(y, g1w, g1b)
        h = conv(h, w1, t)
        h = gn_relu_bf16(h, g2w, g2b)
        h = conv(h, w2, t)
        return gn_final(h, g3w, g3b)

    dt = 1.0 / num_steps

    def rk4_step(i, y):
        t = i.astype(f32) * dt
        k1 = odefunc(t, y)
        acc = y + (dt / 6.0) * k1
        k2 = odefunc(t + 0.5 * dt, y + (0.5 * dt) * k1)
        acc = acc + (dt / 3.0) * k2
        k3 = odefunc(t + 0.5 * dt, y + (0.5 * dt) * k2)
        acc = acc + (dt / 3.0) * k3
        k4 = odefunc(t + dt, y + dt * k3)
        return acc + (dt / 6.0) * k4

    out_ref[0] = lax.fori_loop(0, num_steps, rk4_step, x_ref[0])


def _pack_conv(conv_w, H, W, tpad):
    Cout = conv_w.shape[0]
    Cin = conv_w.shape[1] - 1
    HW = H * W
    wp = jnp.transpose(conv_w[:, 1:], (0, 2, 3, 1)).reshape(Cout, 9 * Cin)
    wt = conv_w[:, 0].reshape(Cout, 9)
    wext = jnp.concatenate(
        [wp, wt, jnp.zeros((Cout, tpad - 9), conv_w.dtype)], axis=1)
    hh = jnp.arange(HW, dtype=jnp.int32) // W
    ww = jnp.arange(HW, dtype=jnp.int32) % W
    valid = []
    for k in range(9):
        dh, dw = k // 3 - 1, k % 3 - 1
        valid.append((hh + dh >= 0) & (hh + dh < H)
                     & (ww + dw >= 0) & (ww + dw < W))
    valid = jnp.stack(valid).astype(jnp.float32)
    valid = jnp.concatenate(
        [valid, jnp.zeros((tpad - 9, HW), jnp.float32)], axis=0)
    return wext, valid


def kernel(x, gn1_w, gn1_b, conv1_w, conv1_b, gn2_w, gn2_b, conv2_w, conv2_b,
           gn3_w, gn3_b):
    B, C, H, W = x.shape
    HW = H * W
    G = _GROUP
    assert B % G == 0

    wp1, valid = _pack_conv(conv1_w, H, W, _TPAD)
    wp2, _ = _pack_conv(conv2_w, H, W, _TPAD)
    wp1 = wp1.astype(jnp.bfloat16)
    wp2 = wp2.astype(jnp.bfloat16)
    valid = valid.astype(jnp.bfloat16)
    zero = jnp.zeros_like(gn1_w)
    cols = jnp.stack([gn1_w, gn1_b, gn2_w, gn2_b, gn3_w, gn3_b,
                      zero, zero], axis=1).astype(jnp.float32)
    cols = jnp.tile(cols, (G, 1))

    xs = x.reshape(B // G, G * C, HW).astype(jnp.float32)

    body = functools.partial(_ode_kernel, spatial_w=W, num_steps=_NUM_STEPS,
                             channels=C)
    y = pl.pallas_call(
        body,
        out_shape=jax.ShapeDtypeStruct((B // G, G * C, HW), jnp.float32),
        grid=(B // G,),
        in_specs=[
            pl.BlockSpec((1, G * C, HW), lambda b: (b, 0, 0)),
            pl.BlockSpec((C, 9 * C + _TPAD), lambda b: (0, 0)),
            pl.BlockSpec((C, 9 * C + _TPAD), lambda b: (0, 0)),
            pl.BlockSpec((_TPAD, HW), lambda b: (0, 0)),
            pl.BlockSpec((G * C, 8), lambda b: (0, 0)),
        ],
        out_specs=pl.BlockSpec((1, G * C, HW), lambda b: (b, 0, 0)),
        compiler_params=pltpu.CompilerParams(
            dimension_semantics=("parallel",)),
    )(xs, wp1, wp2, valid, cols)
    return y.reshape(B, C, H, W)


# speedup vs baseline: 1.0108x; 1.0108x over previous
import functools

import jax
import jax.numpy as jnp
from jax import lax
from jax.experimental import pallas as pl
from jax.experimental.pallas import tpu as pltpu

_EPS = 1e-5
_NUM_STEPS = 8
_GROUP = 4
_TPAD = 16


def _ode_kernel(x_ref, w1_ref, w2_ref, valid_ref, cols_ref, out_ref,
                hb1_ref, hb2_ref, *, spatial_w, num_steps, channels):
    f32 = jnp.float32
    bf16 = jnp.bfloat16
    C = channels
    GC = out_ref.shape[1]
    G = GC // C
    HW = out_ref.shape[2]
    Wd = spatial_w

    w1 = w1_ref[...]
    w2 = w2_ref[...]
    valid = valid_ref[...]
    cols = cols_ref[...]
    g1w, g1b = cols[:, 0:1], cols[:, 1:2]
    g2w, g2b = cols[:, 2:3], cols[:, 3:4]
    g3w, g3b = cols[:, 4:5], cols[:, 5:6]

    wcol = lax.broadcasted_iota(jnp.int32, (1, HW), 1) % Wd
    mask_l = (wcol >= 1).astype(bf16)
    mask_r = (wcol < Wd - 1).astype(bf16)

    def shift(h, off):
        if off == 0:
            return h
        pad = jnp.zeros((h.shape[0], abs(off)), h.dtype)
        if off > 0:
            return jnp.concatenate([h[:, off:], pad], axis=1)
        return jnp.concatenate([pad, h[:, :off]], axis=1)

    def tap_block(hb, trows):
        rows = []
        for dh in (-1, 0, 1):
            base = shift(hb, dh * Wd)
            rows.append(shift(base, -1) * mask_l)
            rows.append(base)
            rows.append(shift(base, 1) * mask_r)
        rows.append(trows)
        return jnp.concatenate(rows, axis=0)

    def gn_relu_bf16(h, gw, gb):
        m = jnp.mean(h, axis=1, keepdims=True)
        ms = jnp.mean(h * h, axis=1, keepdims=True)
        scale = lax.rsqrt((ms - m * m) + _EPS) * gw
        return jnp.maximum(h * scale + (gb - m * scale), 0.0).astype(bf16)

    def gn_final(h, gw, gb):
        m = jnp.mean(h, axis=1, keepdims=True)
        ms = jnp.mean(h * h, axis=1, keepdims=True)
        scale = lax.rsqrt((ms - m * m) + _EPS) * gw
        return h * scale + (gb - m * scale)

    def conv(hb, w, t):
        trows = (t * valid).astype(bf16)
        outs = [jnp.dot(w, tap_block(hb[g * C:(g + 1) * C], trows),
                        preferred_element_type=f32) for g in range(G)]
        return outs[0] if G == 1 else jnp.concatenate(outs, axis=0)

    def odefunc(t, y):
        hb1_ref[...] = gn_relu_bf16(y, g1w, g1b)
        h = conv(hb1_ref[...], w1, t)
        hb2_ref[...] = gn_relu_bf16(h, g2w, g2b)
        h = conv(hb2_ref[...], w2, t)
        return gn_final(h, g3w, g3b)

    dt = 1.0 / num_steps

    def rk4_step(i, y):
        t = i.astype(f32) * dt
        k1 = odefunc(t, y)
        acc = y + (dt / 6.0) * k1
        k2 = odefunc(t + 0.5 * dt, y + (0.5 * dt) * k1)
        acc = acc + (dt / 3.0) * k2
        k3 = odefunc(t + 0.5 * dt, y + (0.5 * dt) * k2)
        acc = acc + (dt / 3.0) * k3
        k4 = odefunc(t + dt, y + dt * k3)
        return acc + (dt / 6.0) * k4

    out_ref[0] = lax.fori_loop(0, num_steps, rk4_step, x_ref[0])


def _pack_conv(conv_w, H, W, tpad):
    Cout = conv_w.shape[0]
    Cin = conv_w.shape[1] - 1
    HW = H * W
    wp = jnp.transpose(conv_w[:, 1:], (0, 2, 3, 1)).reshape(Cout, 9 * Cin)
    wt = conv_w[:, 0].reshape(Cout, 9)
    wext = jnp.concatenate(
        [wp, wt, jnp.zeros((Cout, tpad - 9), conv_w.dtype)], axis=1)
    hh = jnp.arange(HW, dtype=jnp.int32) // W
    ww = jnp.arange(HW, dtype=jnp.int32) % W
    valid = []
    for k in range(9):
        dh, dw = k // 3 - 1, k % 3 - 1
        valid.append((hh + dh >= 0) & (hh + dh < H)
                     & (ww + dw >= 0) & (ww + dw < W))
    valid = jnp.stack(valid).astype(jnp.float32)
    valid = jnp.concatenate(
        [valid, jnp.zeros((tpad - 9, HW), jnp.float32)], axis=0)
    return wext, valid


def kernel(x, gn1_w, gn1_b, conv1_w, conv1_b, gn2_w, gn2_b, conv2_w, conv2_b,
           gn3_w, gn3_b):
    B, C, H, W = x.shape
    HW = H * W
    G = _GROUP
    assert B % G == 0

    wp1, valid = _pack_conv(conv1_w, H, W, _TPAD)
    wp2, _ = _pack_conv(conv2_w, H, W, _TPAD)
    wp1 = wp1.astype(jnp.bfloat16)
    wp2 = wp2.astype(jnp.bfloat16)
    valid = valid.astype(jnp.bfloat16)
    zero = jnp.zeros_like(gn1_w)
    cols = jnp.stack([gn1_w, gn1_b, gn2_w, gn2_b, gn3_w, gn3_b,
                      zero, zero], axis=1).astype(jnp.float32)
    cols = jnp.tile(cols, (G, 1))

    xs = x.reshape(B // G, G * C, HW).astype(jnp.float32)

    body = functools.partial(_ode_kernel, spatial_w=W, num_steps=_NUM_STEPS,
                             channels=C)
    y = pl.pallas_call(
        body,
        out_shape=jax.ShapeDtypeStruct((B // G, G * C, HW), jnp.float32),
        grid=(B // G,),
        in_specs=[
            pl.BlockSpec((1, G * C, HW), lambda b: (b, 0, 0)),
            pl.BlockSpec((C, 9 * C + _TPAD), lambda b: (0, 0)),
            pl.BlockSpec((C, 9 * C + _TPAD), lambda b: (0, 0)),
            pl.BlockSpec((_TPAD, HW), lambda b: (0, 0)),
            pl.BlockSpec((G * C, 8), lambda b: (0, 0)),
        ],
        out_specs=pl.BlockSpec((1, G * C, HW), lambda b: (b, 0, 0)),
        scratch_shapes=[
            pltpu.VMEM((G * C, HW), jnp.bfloat16),
            pltpu.VMEM((G * C, HW), jnp.bfloat16),
        ],
        compiler_params=pltpu.CompilerParams(
            dimension_semantics=("parallel",)),
    )(xs, wp1, wp2, valid, cols)
    return y.reshape(B, C, H, W)
